# baseline (speedup 1.0000x reference)
import functools
import math

import jax
import jax.numpy as jnp
from jax import lax
from jax.experimental import pallas as pl
from jax.experimental.pallas import tpu as pltpu
from jax.experimental.pallas import tpu_sc as plsc

NUM_EXPERTS = 64
TOP_K = 2
HIDDEN = 768
INTER = 3072
CAPACITY_FACTOR = 0.5
BATCH = 2
SEQ = 8192

T = BATCH * SEQ
EC = NUM_EXPERTS - 1
CAP = math.ceil(T / NUM_EXPERTS * CAPACITY_FACTOR)
NROW = EC * CAP
DUMMY = NROW
BUFROWS = NROW + 8

TB = 256
IB = 1536
NIB = INTER // IB


def _router_body(x_ref, wr_ref, rb_ref, rt_ref, cnt_ref, carry_ref):
    pi = pl.program_id(0)

    @pl.when(pi == 0)
    def _():
        carry_ref[...] = jnp.zeros_like(carry_ref)

    xb = x_ref[...]
    logits = jnp.dot(xb, wr_ref[...], preferred_element_type=jnp.float32)
    logits = logits + rb_ref[0:1, :]
    m = jnp.max(logits, axis=-1, keepdims=True)
    p = jnp.exp(logits - m)
    w = p / jnp.sum(p, axis=-1, keepdims=True)

    iota = lax.broadcasted_iota(jnp.int32, (TB, NUM_EXPERTS), 1)
    v0 = jnp.max(w, axis=-1, keepdims=True)
    e0 = jnp.min(jnp.where(w == v0, iota, NUM_EXPERTS), axis=-1, keepdims=True)
    wm = jnp.where(iota == e0, -1.0, w)
    v1 = jnp.max(wm, axis=-1, keepdims=True)
    e1 = jnp.min(jnp.where(wm == v1, iota, NUM_EXPERTS), axis=-1, keepdims=True)

    nw = jnp.where(e0 == EC, v0, jnp.where(e1 == EC, v1, 0.0))
    sel0 = e0 != EC
    sel1 = e1 != EC
    md = (jnp.where((iota == e0) & sel0, 1.0, 0.0)
          + jnp.where((iota == e1) & sel1, 1.0, 0.0))

    ri = lax.broadcasted_iota(jnp.int32, (TB, TB), 0)
    ci = lax.broadcasted_iota(jnp.int32, (TB, TB), 1)
    lstrict = jnp.where(ri > ci, 1.0, 0.0)
    cum = jnp.dot(lstrict, md, preferred_element_type=jnp.float32)
    cum = cum + carry_ref[0:1, :]
    slot0 = jnp.sum(jnp.where(iota == e0, cum, 0.0), axis=-1, keepdims=True)
    slot1 = jnp.sum(jnp.where(iota == e1, cum, 0.0), axis=-1, keepdims=True)
    carry_ref[0:1, :] = carry_ref[0:1, :] + jnp.sum(md, axis=0, keepdims=True)
    cnt_ref[...] = jnp.broadcast_to(carry_ref[0:1, :], cnt_ref.shape)

    r0 = e0.astype(jnp.float32) * CAP + slot0
    r1 = e1.astype(jnp.float32) * CAP + slot1
    val0 = sel0 & (slot0 < CAP)
    val1 = sel1 & (slot1 < CAP)
    d0 = jnp.where(val0, r0, float(DUMMY))
    d1 = jnp.where(val1, r1, float(DUMMY))
    c0 = jnp.where(val0, r0, 0.0)
    c1 = jnp.where(val1, r1, 0.0)
    w0 = jnp.where(val0, v0, 0.0)
    w1 = jnp.where(val1, v1, 0.0)
    rt_ref[...] = jnp.concatenate(
        [d0, d1, c0, c1, w0, w1, nw, jnp.zeros_like(nw)], axis=1)


def _run_router(x2d, wr_t, rb_bcast):
    grid = T // TB
    return pl.pallas_call(
        _router_body,
        grid=(grid,),
        in_specs=[
            pl.BlockSpec((TB, HIDDEN), lambda i: (i, 0)),
            pl.BlockSpec((HIDDEN, NUM_EXPERTS), lambda i: (0, 0)),
            pl.BlockSpec((8, NUM_EXPERTS), lambda i: (0, 0)),
        ],
        out_specs=[
            pl.BlockSpec((TB, 8), lambda i: (i, 0)),
            pl.BlockSpec((8, NUM_EXPERTS), lambda i: (0, 0)),
        ],
        out_shape=[
            jax.ShapeDtypeStruct((T, 8), jnp.float32),
            jax.ShapeDtypeStruct((8, NUM_EXPERTS), jnp.float32),
        ],
        scratch_shapes=[pltpu.VMEM((8, NUM_EXPERTS), jnp.float32)],
        compiler_params=pltpu.CompilerParams(
            dimension_semantics=("arbitrary",)),
    )(x2d, wr_t, rb_bcast)


_NW = 32
_TOK_PER_W = T // _NW
_CHUNK = 128
_NCH = _TOK_PER_W // _CHUNK


@functools.cache
def _make_dispatch_sc():
    mesh = plsc.VectorSubcoreMesh(core_axis_name="c", subcore_axis_name="s")

    @functools.partial(
        pl.kernel,
        mesh=mesh,
        out_type=jax.ShapeDtypeStruct((BUFROWS, HIDDEN), jnp.float32),
        scratch_types=[
            pltpu.VMEM((_CHUNK, HIDDEN), jnp.float32),
            pltpu.VMEM((_CHUNK,), jnp.int32),
            pltpu.VMEM((_CHUNK,), jnp.int32),
            pltpu.SemaphoreType.DMA,
        ],
    )
    def _dispatch_sc(x_hbm, d0_hbm, d1_hbm, buf_hbm, xbuf, idx0, idx1, sem):
        wid = lax.axis_index("s") * 2 + lax.axis_index("c")
        for c in range(_NCH):
            base = wid * _TOK_PER_W + c * _CHUNK
            pltpu.sync_copy(x_hbm.at[pl.ds(base, _CHUNK)], xbuf)
            pltpu.sync_copy(d0_hbm.at[pl.ds(base, _CHUNK)], idx0)
            pltpu.sync_copy(d1_hbm.at[pl.ds(base, _CHUNK)], idx1)
            pltpu.async_copy(xbuf, buf_hbm.at[idx0], sem).wait()
            pltpu.async_copy(xbuf, buf_hbm.at[idx1], sem).wait()

    return _dispatch_sc


def _ffn_body(x_ref, vm_ref, w1_ref, w2_ref, out_ref):
    ib = pl.program_id(1)
    xb = jnp.where(vm_ref[...] > 0, x_ref[...], 0.0)
    h = jnp.dot(xb, w1_ref[0], preferred_element_type=jnp.float32)
    h = jnp.maximum(h, 0.0)
    o = jnp.dot(h, w2_ref[0], preferred_element_type=jnp.float32)

    @pl.when(ib == 0)
    def _():
        out_ref[...] = o

    @pl.when(ib != 0)
    def _():
        out_ref[...] = out_ref[...] + o


def _run_ffn(buf, vmask, w1, w2):
    return pl.pallas_call(
        _ffn_body,
        grid=(EC, NIB),
        in_specs=[
            pl.BlockSpec((CAP, HIDDEN), lambda e, i: (e, 0)),
            pl.BlockSpec((CAP, 1), lambda e, i: (e, 0)),
            pl.BlockSpec((1, HIDDEN, IB), lambda e, i: (e, 0, i)),
            pl.BlockSpec((1, IB, HIDDEN), lambda e, i: (e, i, 0)),
        ],
        out_specs=pl.BlockSpec((CAP, HIDDEN), lambda e, i: (e, 0)),
        out_shape=jax.ShapeDtypeStruct((NROW, HIDDEN), jnp.float32),
        compiler_params=pltpu.CompilerParams(
            dimension_semantics=("arbitrary", "arbitrary")),
    )(buf, vmask, w1, w2)


@functools.cache
def _make_gather_sc():
    mesh = plsc.VectorSubcoreMesh(core_axis_name="c", subcore_axis_name="s")

    @functools.partial(
        pl.kernel,
        mesh=mesh,
        out_type=[
            jax.ShapeDtypeStruct((T, HIDDEN), jnp.float32),
            jax.ShapeDtypeStruct((T, HIDDEN), jnp.float32),
        ],
        scratch_types=[
            pltpu.VMEM((_CHUNK, HIDDEN), jnp.float32),
            pltpu.VMEM((_CHUNK,), jnp.int32),
            pltpu.SemaphoreType.DMA,
        ],
    )
    def _gather_sc(out_hbm, c0_hbm, c1_hbm, g0_hbm, g1_hbm, gbuf, idx, sem):
        wid = lax.axis_index("s") * 2 + lax.axis_index("c")
        for c in range(_NCH):
            base = wid * _TOK_PER_W + c * _CHUNK
            pltpu.sync_copy(c0_hbm.at[pl.ds(base, _CHUNK)], idx)
            pltpu.async_copy(out_hbm.at[idx], gbuf, sem).wait()
            pltpu.sync_copy(gbuf, g0_hbm.at[pl.ds(base, _CHUNK)])
            pltpu.sync_copy(c1_hbm.at[pl.ds(base, _CHUNK)], idx)
            pltpu.async_copy(out_hbm.at[idx], gbuf, sem).wait()
            pltpu.sync_copy(gbuf, g1_hbm.at[pl.ds(base, _CHUNK)])

    return _gather_sc


def _combine_body(g0_ref, g1_ref, x_ref, rt_ref, y_ref):
    w0 = rt_ref[:, 4:5]
    w1 = rt_ref[:, 5:6]
    nw = rt_ref[:, 6:7]
    y_ref[...] = (w0 * g0_ref[...] + w1 * g1_ref[...] + nw * x_ref[...])


def _run_combine(g0, g1, x2d, rt):
    grid = T // TB
    return pl.pallas_call(
        _combine_body,
        grid=(grid,),
        in_specs=[
            pl.BlockSpec((TB, HIDDEN), lambda i: (i, 0)),
            pl.BlockSpec((TB, HIDDEN), lambda i: (i, 0)),
            pl.BlockSpec((TB, HIDDEN), lambda i: (i, 0)),
            pl.BlockSpec((TB, 8), lambda i: (i, 0)),
        ],
        out_specs=pl.BlockSpec((TB, HIDDEN), lambda i: (i, 0)),
        out_shape=jax.ShapeDtypeStruct((T, HIDDEN), jnp.float32),
        compiler_params=pltpu.CompilerParams(
            dimension_semantics=("arbitrary",)),
    )(g0, g1, x2d, rt)


def kernel(x, experts_inter, experts_out, router_w, router_b):
    x2d = x.reshape(T, HIDDEN)
    wr_t = router_w.T
    rb_bcast = jnp.broadcast_to(router_b[None, :], (8, NUM_EXPERTS))

    rt, cnt = _run_router(x2d, wr_t, rb_bcast)

    d0 = rt[:, 0].astype(jnp.int32)
    d1 = rt[:, 1].astype(jnp.int32)
    c0 = rt[:, 2].astype(jnp.int32)
    c1 = rt[:, 3].astype(jnp.int32)

    counts = cnt[0, :EC]
    slot_iota = lax.broadcasted_iota(jnp.float32, (EC, CAP), 1)
    vmask = (slot_iota < counts[:, None]).astype(jnp.float32).reshape(NROW, 1)

    buf = _make_dispatch_sc()(x2d, d0, d1)
    out_buf = _run_ffn(buf, vmask, experts_inter, experts_out)
    g0, g1 = _make_gather_sc()(out_buf, c0, c1)
    y = _run_combine(g0, g1, x2d, rt)
    return y.reshape(BATCH, SEQ, HIDDEN)


# speedup vs baseline: 1.5569x; 1.5569x over previous
import functools
import math

import jax
import jax.numpy as jnp
from jax import lax
from jax.experimental import pallas as pl
from jax.experimental.pallas import tpu as pltpu
from jax.experimental.pallas import tpu_sc as plsc

NUM_EXPERTS = 64
TOP_K = 2
HIDDEN = 768
INTER = 3072
CAPACITY_FACTOR = 0.5
BATCH = 2
SEQ = 8192

T = BATCH * SEQ
EC = NUM_EXPERTS - 1
CAP = math.ceil(T / NUM_EXPERTS * CAPACITY_FACTOR)
NROW = EC * CAP
DUMMY = NROW
GROWS = 8192
TOKP = 8208

TB = 256

_NW = 32
_D1T = 4
_D1TOK = T // _D1T


def _sc_mesh():
    return plsc.VectorSubcoreMesh(core_axis_name="c", subcore_axis_name="s")


def _router_body(x_ref, wr_ref, rb_ref, rt_ref, cnt_ref, carry_ref):
    pi = pl.program_id(0)

    @pl.when(pi == 0)
    def _():
        carry_ref[...] = jnp.zeros_like(carry_ref)

    xb = x_ref[...]
    logits = jnp.dot(xb, wr_ref[...], preferred_element_type=jnp.float32)
    logits = logits + rb_ref[0:1, :]
    m = jnp.max(logits, axis=-1, keepdims=True)
    p = jnp.exp(logits - m)
    w = p / jnp.sum(p, axis=-1, keepdims=True)

    iota = lax.broadcasted_iota(jnp.int32, (TB, NUM_EXPERTS), 1)
    v0 = jnp.max(w, axis=-1, keepdims=True)
    e0 = jnp.min(jnp.where(w == v0, iota, NUM_EXPERTS), axis=-1, keepdims=True)
    wm = jnp.where(iota == e0, -1.0, w)
    v1 = jnp.max(wm, axis=-1, keepdims=True)
    e1 = jnp.min(jnp.where(wm == v1, iota, NUM_EXPERTS), axis=-1, keepdims=True)

    nw = jnp.where(e0 == EC, v0, jnp.where(e1 == EC, v1, 0.0))
    sel0 = e0 != EC
    sel1 = e1 != EC
    md = (jnp.where((iota == e0) & sel0, 1.0, 0.0)
          + jnp.where((iota == e1) & sel1, 1.0, 0.0))

    ri = lax.broadcasted_iota(jnp.int32, (TB, TB), 0)
    ci = lax.broadcasted_iota(jnp.int32, (TB, TB), 1)
    lstrict = jnp.where(ri > ci, 1.0, 0.0)
    cum = jnp.dot(lstrict, md, preferred_element_type=jnp.float32)
    cum = cum + carry_ref[0:1, :]
    slot0 = jnp.sum(jnp.where(iota == e0, cum, 0.0), axis=-1, keepdims=True)
    slot1 = jnp.sum(jnp.where(iota == e1, cum, 0.0), axis=-1, keepdims=True)
    carry_ref[0:1, :] = carry_ref[0:1, :] + jnp.sum(md, axis=0, keepdims=True)
    cnt_ref[...] = jnp.broadcast_to(carry_ref[0:1, :], cnt_ref.shape)

    r0 = e0.astype(jnp.float32) * CAP + slot0
    r1 = e1.astype(jnp.float32) * CAP + slot1
    val0 = sel0 & (slot0 < CAP)
    val1 = sel1 & (slot1 < CAP)
    d0 = jnp.where(val0, r0, float(DUMMY))
    d1 = jnp.where(val1, r1, float(DUMMY))
    c0 = jnp.where(val0, r0, 0.0)
    c1 = jnp.where(val1, r1, 0.0)
    w0 = jnp.where(val0, v0, 0.0)
    w1 = jnp.where(val1, v1, 0.0)
    rt_ref[...] = jnp.concatenate(
        [d0, d1, c0, c1, w0, w1, nw, jnp.zeros_like(nw)], axis=1)


def _run_router(x2d, wr_t, rb_bcast):
    grid = T // TB
    return pl.pallas_call(
        _router_body,
        grid=(grid,),
        in_specs=[
            pl.BlockSpec((TB, HIDDEN), lambda i: (i, 0)),
            pl.BlockSpec((HIDDEN, NUM_EXPERTS), lambda i: (0, 0)),
            pl.BlockSpec((8, NUM_EXPERTS), lambda i: (0, 0)),
        ],
        out_specs=[
            pl.BlockSpec((TB, 8), lambda i: (i, 0)),
            pl.BlockSpec((8, NUM_EXPERTS), lambda i: (0, 0)),
        ],
        out_shape=[
            jax.ShapeDtypeStruct((T, 8), jnp.float32),
            jax.ShapeDtypeStruct((8, NUM_EXPERTS), jnp.float32),
        ],
        scratch_shapes=[pltpu.VMEM((8, NUM_EXPERTS), jnp.float32)],
        compiler_params=pltpu.CompilerParams(
            dimension_semantics=("arbitrary",)),
    )(x2d, wr_t, rb_bcast)


@functools.cache
def _make_tokmap_sc():
    @functools.partial(
        pl.kernel,
        mesh=_sc_mesh(),
        out_type=jax.ShapeDtypeStruct((_D1T, TOKP), jnp.int32),
        scratch_types=[
            pltpu.VMEM((2 * _D1TOK,), jnp.int32),
            pltpu.VMEM((TOKP,), jnp.int32),
        ],
        compiler_params=pltpu.CompilerParams(needs_layout_passes=False),
    )
    def _tokmap(d0_hbm, d1_hbm, tokp_hbm, dbuf, tokbuf):
        wid = lax.axis_index("s") * 2 + lax.axis_index("c")

        @pl.when(wid < _D1T)
        def _():
            def zb(i, carry):
                tokbuf[pl.ds(i * 16, 16)] = jnp.zeros((16,), jnp.int32)
                return carry

            lax.fori_loop(0, TOKP // 16, zb, 0)
            base = wid * _D1TOK
            pltpu.sync_copy(d0_hbm.at[pl.ds(base, _D1TOK)],
                            dbuf.at[pl.ds(0, _D1TOK)])
            pltpu.sync_copy(d1_hbm.at[pl.ds(base, _D1TOK)],
                            dbuf.at[pl.ds(_D1TOK, _D1TOK)])
            iota16 = lax.iota(jnp.int32, 16)

            def sc0(i, carry):
                idx = dbuf[pl.ds(i * 16, 16)]
                plsc.store_scatter(tokbuf, [idx], base + i * 16 + iota16)
                return carry

            lax.fori_loop(0, _D1TOK // 16, sc0, 0)

            def sc1(i, carry):
                idx = dbuf[pl.ds(_D1TOK + i * 16, 16)]
                plsc.store_scatter(tokbuf, [idx], base + i * 16 + iota16)
                return carry

            lax.fori_loop(0, _D1TOK // 16, sc1, 0)
            pltpu.sync_copy(tokbuf, tokp_hbm.at[wid])

    return _tokmap


_D2CH = 64
_D2NJ = GROWS // _NW // _D2CH


@functools.cache
def _make_dispatch_sc():
    @functools.partial(
        pl.kernel,
        mesh=_sc_mesh(),
        out_type=jax.ShapeDtypeStruct((GROWS, HIDDEN), jnp.float32),
        scratch_types=[
            pltpu.VMEM((_D1T, GROWS // _NW), jnp.int32),
            pltpu.VMEM((GROWS // _NW,), jnp.int32),
            pltpu.VMEM((_D2CH, HIDDEN), jnp.float32),
            pltpu.VMEM((_D2CH, HIDDEN), jnp.float32),
            pltpu.SemaphoreType.DMA,
            pltpu.SemaphoreType.DMA,
            pltpu.SemaphoreType.DMA,
            pltpu.SemaphoreType.DMA,
        ],
    )
    def _dispatch(x_hbm, tokp_hbm, buf_hbm, tokc, isum, xr0, xr1,
                  g0s, g1s, w0s, w1s):
        wid = lax.axis_index("s") * 2 + lax.axis_index("c")
        xr = [xr0, xr1]
        gs = [g0s, g1s]
        ws = [w0s, w1s]
        rows_per_w = GROWS // _NW

        rbase = wid * rows_per_w
        pltpu.sync_copy(tokp_hbm.at[:, pl.ds(rbase, rows_per_w)], tokc)
        for g in range(rows_per_w // 16):
            sl = pl.ds(g * 16, 16)
            s = tokc[0, sl] + tokc[1, sl] + tokc[2, sl] + tokc[3, sl]
            isum[sl] = jnp.minimum(s, T - 1)

        def start_gather(j, b):
            return pltpu.async_copy(
                x_hbm.at[isum.at[pl.ds(j * _D2CH, _D2CH)]], xr[b], gs[b])

        gds = [None, None]
        wds = [None, None]
        gds[0] = start_gather(0, 0)
        for j in range(_D2NJ):
            b = j % 2
            gds[b].wait()
            if j + 1 < _D2NJ:
                if j >= 1:
                    wds[1 - b].wait()
                gds[1 - b] = start_gather(j + 1, 1 - b)
            wds[b] = pltpu.async_copy(
                xr[b], buf_hbm.at[pl.ds(rbase + j * _D2CH, _D2CH)], ws[b])
        wds[(_D2NJ - 1) % 2].wait()
        if _D2NJ >= 2:
            wds[_D2NJ % 2].wait()

    return _dispatch


def _ffn_body(x_ref, vm_ref, w1_ref, w2_ref, out_ref):
    xb = jnp.where(vm_ref[...] > 0, x_ref[...], 0.0)
    h = jnp.dot(xb, w1_ref[0], preferred_element_type=jnp.float32)
    h = jnp.maximum(h, 0.0)
    out_ref[...] = jnp.dot(h, w2_ref[0], preferred_element_type=jnp.float32)


def _run_ffn(buf, vmask, w1, w2):
    return pl.pallas_call(
        _ffn_body,
        grid=(EC,),
        in_specs=[
            pl.BlockSpec((CAP, HIDDEN), lambda e: (e, 0)),
            pl.BlockSpec((CAP, 1), lambda e: (e, 0)),
            pl.BlockSpec((1, HIDDEN, INTER), lambda e: (e, 0, 0)),
            pl.BlockSpec((1, INTER, HIDDEN), lambda e: (e, 0, 0)),
        ],
        out_specs=pl.BlockSpec((CAP, HIDDEN), lambda e: (e, 0)),
        out_shape=jax.ShapeDtypeStruct((NROW, HIDDEN), jnp.float32),
        compiler_params=pltpu.CompilerParams(
            dimension_semantics=("arbitrary",)),
    )(buf, vmask, w1, w2)


_CCH = 64
_CNJ = (T // _NW // _CCH) * 2


@functools.cache
def _make_gather_sc():
    @functools.partial(
        pl.kernel,
        mesh=_sc_mesh(),
        out_type=[
            jax.ShapeDtypeStruct((T, HIDDEN), jnp.float32),
            jax.ShapeDtypeStruct((T, HIDDEN), jnp.float32),
        ],
        scratch_types=[
            pltpu.VMEM((T // _NW // _CCH, _CCH), jnp.int32),
            pltpu.VMEM((T // _NW // _CCH, _CCH), jnp.int32),
            pltpu.VMEM((_CCH, HIDDEN), jnp.float32),
            pltpu.VMEM((_CCH, HIDDEN), jnp.float32),
            pltpu.SemaphoreType.DMA,
            pltpu.SemaphoreType.DMA,
            pltpu.SemaphoreType.DMA,
            pltpu.SemaphoreType.DMA,
        ],
    )
    def _gather(out_hbm, c0_hbm, c1_hbm, g0_hbm, g1_hbm,
                idx0s, idx1s, gb0, gb1, g0sem, g1sem, w0sem, w1sem):
        wid = lax.axis_index("s") * 2 + lax.axis_index("c")
        gb = [gb0, gb1]
        gsems = [g0sem, g1sem]
        wsems = [w0sem, w1sem]
        nchunk = T // _NW // _CCH
        pltpu.sync_copy(c0_hbm.at[wid], idx0s)
        pltpu.sync_copy(c1_hbm.at[wid], idx1s)

        def job(j):
            k, c = j // nchunk, j % nchunk
            idxs = idx0s if k == 0 else idx1s
            dst = g0_hbm if k == 0 else g1_hbm
            tbase = wid * (T // _NW) + c * _CCH
            return idxs.at[c], dst.at[pl.ds(tbase, _CCH)]

        gds = [None, None]
        wds = [None, None]
        isrc, _ = job(0)
        gds[0] = pltpu.async_copy(out_hbm.at[isrc], gb[0], gsems[0])
        for j in range(_CNJ):
            b = j % 2
            gds[b].wait()
            if j + 1 < _CNJ:
                if j >= 1:
                    wds[1 - b].wait()
                isrc, _ = job(j + 1)
                gds[1 - b] = pltpu.async_copy(
                    out_hbm.at[isrc], gb[1 - b], gsems[1 - b])
            _, dst = job(j)
            wds[b] = pltpu.async_copy(gb[b], dst, wsems[b])
        wds[(_CNJ - 1) % 2].wait()
        wds[_CNJ % 2].wait()

    return _gather


def _combine_body(g0_ref, g1_ref, x_ref, rt_ref, y_ref):
    w0 = rt_ref[:, 4:5]
    w1 = rt_ref[:, 5:6]
    nw = rt_ref[:, 6:7]
    y_ref[...] = (w0 * g0_ref[...] + w1 * g1_ref[...] + nw * x_ref[...])


def _run_combine(g0, g1, x2d, rt):
    grid = T // TB
    return pl.pallas_call(
        _combine_body,
        grid=(grid,),
        in_specs=[
            pl.BlockSpec((TB, HIDDEN), lambda i: (i, 0)),
            pl.BlockSpec((TB, HIDDEN), lambda i: (i, 0)),
            pl.BlockSpec((TB, HIDDEN), lambda i: (i, 0)),
            pl.BlockSpec((TB, 8), lambda i: (i, 0)),
        ],
        out_specs=pl.BlockSpec((TB, HIDDEN), lambda i: (i, 0)),
        out_shape=jax.ShapeDtypeStruct((T, HIDDEN), jnp.float32),
        compiler_params=pltpu.CompilerParams(
            dimension_semantics=("arbitrary",)),
    )(g0, g1, x2d, rt)


def kernel(x, experts_inter, experts_out, router_w, router_b):
    x2d = x.reshape(T, HIDDEN)
    wr_t = router_w.T
    rb_bcast = jnp.broadcast_to(router_b[None, :], (8, NUM_EXPERTS))

    rt, cnt = _run_router(x2d, wr_t, rb_bcast)

    d0 = rt[:, 0].astype(jnp.int32)
    d1 = rt[:, 1].astype(jnp.int32)
    c0r = rt[:, 2].astype(jnp.int32).reshape(_NW, T // _NW // _CCH, _CCH)
    c1r = rt[:, 3].astype(jnp.int32).reshape(_NW, T // _NW // _CCH, _CCH)

    counts = cnt[0, :EC]
    slot_iota = lax.broadcasted_iota(jnp.float32, (EC, CAP), 1)
    vmask = (slot_iota < counts[:, None]).astype(jnp.float32).reshape(NROW, 1)

    tokp = _make_tokmap_sc()(d0, d1)
    buf = _make_dispatch_sc()(x2d, tokp)
    out_buf = _run_ffn(buf, vmask, experts_inter, experts_out)
    g0, g1 = _make_gather_sc()(out_buf, c0r, c1r)
    y = _run_combine(g0, g1, x2d, rt)
    return y.reshape(BATCH, SEQ, HIDDEN)


# speedup vs baseline: 3.9300x; 2.5242x over previous
import functools
import math

import jax
import jax.numpy as jnp
from jax import lax
from jax.experimental import pallas as pl
from jax.experimental.pallas import tpu as pltpu
from jax.experimental.pallas import tpu_sc as plsc

NUM_EXPERTS = 64
TOP_K = 2
HIDDEN = 768
INTER = 3072
CAPACITY_FACTOR = 0.5
BATCH = 2
SEQ = 8192

T = BATCH * SEQ
EC = NUM_EXPERTS - 1
CAP = math.ceil(T / NUM_EXPERTS * CAPACITY_FACTOR)
NROW = EC * CAP
DUMMY = NROW
GROWS = 8192
TOKP = 8208

TB = 256

_NW = 32
_D1T = 4
_D1TOK = T // _D1T


def _sc_mesh():
    return plsc.VectorSubcoreMesh(core_axis_name="c", subcore_axis_name="s")


def _router_body(x_ref, wr_ref, rb_ref, rt_ref, cnt_ref, carry_ref):
    pi = pl.program_id(0)

    @pl.when(pi == 0)
    def _():
        carry_ref[...] = jnp.zeros_like(carry_ref)

    xb = x_ref[...]
    logits = jnp.dot(xb, wr_ref[...], preferred_element_type=jnp.float32)
    logits = logits + rb_ref[0:1, :]
    m = jnp.max(logits, axis=-1, keepdims=True)
    p = jnp.exp(logits - m)
    w = p / jnp.sum(p, axis=-1, keepdims=True)

    iota = lax.broadcasted_iota(jnp.int32, (TB, NUM_EXPERTS), 1)
    v0 = jnp.max(w, axis=-1, keepdims=True)
    e0 = jnp.min(jnp.where(w == v0, iota, NUM_EXPERTS), axis=-1, keepdims=True)
    wm = jnp.where(iota == e0, -1.0, w)
    v1 = jnp.max(wm, axis=-1, keepdims=True)
    e1 = jnp.min(jnp.where(wm == v1, iota, NUM_EXPERTS), axis=-1, keepdims=True)

    nw = jnp.where(e0 == EC, v0, jnp.where(e1 == EC, v1, 0.0))
    sel0 = e0 != EC
    sel1 = e1 != EC
    md = (jnp.where((iota == e0) & sel0, 1.0, 0.0)
          + jnp.where((iota == e1) & sel1, 1.0, 0.0))

    ri = lax.broadcasted_iota(jnp.int32, (TB, TB), 0)
    ci = lax.broadcasted_iota(jnp.int32, (TB, TB), 1)
    lstrict = jnp.where(ri > ci, 1.0, 0.0)
    cum = jnp.dot(lstrict, md, preferred_element_type=jnp.float32)
    cum = cum + carry_ref[0:1, :]
    slot0 = jnp.sum(jnp.where(iota == e0, cum, 0.0), axis=-1, keepdims=True)
    slot1 = jnp.sum(jnp.where(iota == e1, cum, 0.0), axis=-1, keepdims=True)
    carry_ref[0:1, :] = carry_ref[0:1, :] + jnp.sum(md, axis=0, keepdims=True)
    cnt_ref[...] = jnp.broadcast_to(carry_ref[0:1, :], cnt_ref.shape)

    r0 = e0.astype(jnp.float32) * CAP + slot0
    r1 = e1.astype(jnp.float32) * CAP + slot1
    val0 = sel0 & (slot0 < CAP)
    val1 = sel1 & (slot1 < CAP)
    d0 = jnp.where(val0, r0, float(DUMMY))
    d1 = jnp.where(val1, r1, float(DUMMY))
    ti = (pl.program_id(0) * TB
          + lax.broadcasted_iota(jnp.int32, (TB, 1), 0)) % NROW
    tif = ti.astype(jnp.float32)
    c0 = jnp.where(val0, r0, tif)
    c1 = jnp.where(val1, r1, tif)
    w0 = jnp.where(val0, v0, 0.0)
    w1 = jnp.where(val1, v1, 0.0)
    rt_ref[...] = jnp.concatenate(
        [d0, d1, c0, c1, w0, w1, nw, jnp.zeros_like(nw)], axis=1)


def _run_router(x2d, wr_t, rb_bcast):
    grid = T // TB
    return pl.pallas_call(
        _router_body,
        grid=(grid,),
        in_specs=[
            pl.BlockSpec((TB, HIDDEN), lambda i: (i, 0)),
            pl.BlockSpec((HIDDEN, NUM_EXPERTS), lambda i: (0, 0)),
            pl.BlockSpec((8, NUM_EXPERTS), lambda i: (0, 0)),
        ],
        out_specs=[
            pl.BlockSpec((TB, 8), lambda i: (i, 0)),
            pl.BlockSpec((8, NUM_EXPERTS), lambda i: (0, 0)),
        ],
        out_shape=[
            jax.ShapeDtypeStruct((T, 8), jnp.float32),
            jax.ShapeDtypeStruct((8, NUM_EXPERTS), jnp.float32),
        ],
        scratch_shapes=[pltpu.VMEM((8, NUM_EXPERTS), jnp.float32)],
        compiler_params=pltpu.CompilerParams(
            dimension_semantics=("arbitrary",)),
    )(x2d, wr_t, rb_bcast)


@functools.cache
def _make_tokmap_sc():
    @functools.partial(
        pl.kernel,
        mesh=_sc_mesh(),
        out_type=jax.ShapeDtypeStruct((_D1T, TOKP), jnp.int32),
        scratch_types=[
            pltpu.VMEM((2 * _D1TOK,), jnp.int32),
            pltpu.VMEM((TOKP,), jnp.int32),
        ],
        compiler_params=pltpu.CompilerParams(needs_layout_passes=False),
    )
    def _tokmap(d0_hbm, d1_hbm, tokp_hbm, dbuf, tokbuf):
        wid = lax.axis_index("s") * 2 + lax.axis_index("c")

        @pl.when(wid < _D1T)
        def _():
            def zb(i, carry):
                tokbuf[pl.ds(i * 16, 16)] = jnp.zeros((16,), jnp.int32)
                return carry

            lax.fori_loop(0, TOKP // 16, zb, 0)
            base = wid * _D1TOK
            pltpu.sync_copy(d0_hbm.at[pl.ds(base, _D1TOK)],
                            dbuf.at[pl.ds(0, _D1TOK)])
            pltpu.sync_copy(d1_hbm.at[pl.ds(base, _D1TOK)],
                            dbuf.at[pl.ds(_D1TOK, _D1TOK)])
            iota16 = lax.iota(jnp.int32, 16)

            def sc0(i, carry):
                idx = dbuf[pl.ds(i * 16, 16)]
                plsc.store_scatter(tokbuf, [idx], base + i * 16 + iota16)
                return carry

            lax.fori_loop(0, _D1TOK // 16, sc0, 0)

            def sc1(i, carry):
                idx = dbuf[pl.ds(_D1TOK + i * 16, 16)]
                plsc.store_scatter(tokbuf, [idx], base + i * 16 + iota16)
                return carry

            lax.fori_loop(0, _D1TOK // 16, sc1, 0)
            pltpu.sync_copy(tokbuf, tokp_hbm.at[wid])

    return _tokmap


_D2CH = 64
_D2NJ = GROWS // _NW // _D2CH


@functools.cache
def _make_dispatch_sc():
    @functools.partial(
        pl.kernel,
        mesh=_sc_mesh(),
        out_type=jax.ShapeDtypeStruct((GROWS, HIDDEN), jnp.float32),
        scratch_types=[
            pltpu.VMEM((_D1T, GROWS // _NW), jnp.int32),
            pltpu.VMEM((GROWS // _NW,), jnp.int32),
            pltpu.VMEM((_D2CH, HIDDEN), jnp.float32),
            pltpu.VMEM((_D2CH, HIDDEN), jnp.float32),
            pltpu.SemaphoreType.DMA,
            pltpu.SemaphoreType.DMA,
            pltpu.SemaphoreType.DMA,
            pltpu.SemaphoreType.DMA,
        ],
    )
    def _dispatch(x_hbm, tokp_hbm, buf_hbm, tokc, isum, xr0, xr1,
                  g0s, g1s, w0s, w1s):
        wid = lax.axis_index("s") * 2 + lax.axis_index("c")
        xr = [xr0, xr1]
        gs = [g0s, g1s]
        ws = [w0s, w1s]
        rows_per_w = GROWS // _NW

        rbase = wid * rows_per_w
        pltpu.sync_copy(tokp_hbm.at[:, pl.ds(rbase, rows_per_w)], tokc)
        for g in range(rows_per_w // 16):
            sl = pl.ds(g * 16, 16)
            s = tokc[0, sl] + tokc[1, sl] + tokc[2, sl] + tokc[3, sl]
            isum[sl] = jnp.minimum(s, T - 1)

        def start_gather(j, b):
            return pltpu.async_copy(
                x_hbm.at[isum.at[pl.ds(j * _D2CH, _D2CH)]], xr[b], gs[b])

        gds = [None, None]
        wds = [None, None]
        gds[0] = start_gather(0, 0)
        for j in range(_D2NJ):
            b = j % 2
            gds[b].wait()
            if j + 1 < _D2NJ:
                if j >= 1:
                    wds[1 - b].wait()
                gds[1 - b] = start_gather(j + 1, 1 - b)
            wds[b] = pltpu.async_copy(
                xr[b], buf_hbm.at[pl.ds(rbase + j * _D2CH, _D2CH)], ws[b])
        wds[(_D2NJ - 1) % 2].wait()
        if _D2NJ >= 2:
            wds[_D2NJ % 2].wait()

    return _dispatch


def _ffn_body(x_ref, vm_ref, w1_ref, w2_ref, out_ref):
    xb = jnp.where(vm_ref[...] > 0, x_ref[...], 0.0)
    h = jnp.dot(xb, w1_ref[0], preferred_element_type=jnp.float32)
    h = jnp.maximum(h, 0.0)
    out_ref[...] = jnp.dot(h, w2_ref[0], preferred_element_type=jnp.float32)


def _run_ffn(buf, vmask, w1, w2):
    return pl.pallas_call(
        _ffn_body,
        grid=(EC,),
        in_specs=[
            pl.BlockSpec((CAP, HIDDEN), lambda e: (e, 0)),
            pl.BlockSpec((CAP, 1), lambda e: (e, 0)),
            pl.BlockSpec((1, HIDDEN, INTER), lambda e: (e, 0, 0)),
            pl.BlockSpec((1, INTER, HIDDEN), lambda e: (e, 0, 0)),
        ],
        out_specs=pl.BlockSpec((CAP, HIDDEN), lambda e: (e, 0)),
        out_shape=jax.ShapeDtypeStruct((NROW, HIDDEN), jnp.float32),
        compiler_params=pltpu.CompilerParams(
            dimension_semantics=("arbitrary",)),
    )(buf, vmask, w1, w2)


_CCH = 64
_CNJ = (T // _NW // _CCH) * 2


@functools.cache
def _make_gather_sc():
    @functools.partial(
        pl.kernel,
        mesh=_sc_mesh(),
        out_type=[
            jax.ShapeDtypeStruct((T, HIDDEN), jnp.float32),
            jax.ShapeDtypeStruct((T, HIDDEN), jnp.float32),
        ],
        scratch_types=[
            pltpu.VMEM((T // _NW // _CCH, _CCH), jnp.int32),
            pltpu.VMEM((T // _NW // _CCH, _CCH), jnp.int32),
            pltpu.VMEM((_CCH, HIDDEN), jnp.float32),
            pltpu.VMEM((_CCH, HIDDEN), jnp.float32),
            pltpu.SemaphoreType.DMA,
            pltpu.SemaphoreType.DMA,
            pltpu.SemaphoreType.DMA,
            pltpu.SemaphoreType.DMA,
        ],
    )
    def _gather(out_hbm, c0_hbm, c1_hbm, g0_hbm, g1_hbm,
                idx0s, idx1s, gb0, gb1, g0sem, g1sem, w0sem, w1sem):
        wid = lax.axis_index("s") * 2 + lax.axis_index("c")
        gb = [gb0, gb1]
        gsems = [g0sem, g1sem]
        wsems = [w0sem, w1sem]
        nchunk = T // _NW // _CCH
        pltpu.sync_copy(c0_hbm.at[wid], idx0s)
        pltpu.sync_copy(c1_hbm.at[wid], idx1s)

        def job(j):
            k, c = j // nchunk, j % nchunk
            idxs = idx0s if k == 0 else idx1s
            dst = g0_hbm if k == 0 else g1_hbm
            tbase = wid * (T // _NW) + c * _CCH
            return idxs.at[c], dst.at[pl.ds(tbase, _CCH)]

        gds = [None, None]
        wds = [None, None]
        isrc, _ = job(0)
        gds[0] = pltpu.async_copy(out_hbm.at[isrc], gb[0], gsems[0])
        for j in range(_CNJ):
            b = j % 2
            gds[b].wait()
            if j + 1 < _CNJ:
                if j >= 1:
                    wds[1 - b].wait()
                isrc, _ = job(j + 1)
                gds[1 - b] = pltpu.async_copy(
                    out_hbm.at[isrc], gb[1 - b], gsems[1 - b])
            _, dst = job(j)
            wds[b] = pltpu.async_copy(gb[b], dst, wsems[b])
        wds[(_CNJ - 1) % 2].wait()
        wds[_CNJ % 2].wait()

    return _gather


def _combine_body(g0_ref, g1_ref, x_ref, rt_ref, y_ref):
    w0 = rt_ref[:, 4:5]
    w1 = rt_ref[:, 5:6]
    nw = rt_ref[:, 6:7]
    y_ref[...] = (w0 * g0_ref[...] + w1 * g1_ref[...] + nw * x_ref[...])


def _run_combine(g0, g1, x2d, rt):
    grid = T // TB
    return pl.pallas_call(
        _combine_body,
        grid=(grid,),
        in_specs=[
            pl.BlockSpec((TB, HIDDEN), lambda i: (i, 0)),
            pl.BlockSpec((TB, HIDDEN), lambda i: (i, 0)),
            pl.BlockSpec((TB, HIDDEN), lambda i: (i, 0)),
            pl.BlockSpec((TB, 8), lambda i: (i, 0)),
        ],
        out_specs=pl.BlockSpec((TB, HIDDEN), lambda i: (i, 0)),
        out_shape=jax.ShapeDtypeStruct((T, HIDDEN), jnp.float32),
        compiler_params=pltpu.CompilerParams(
            dimension_semantics=("arbitrary",)),
    )(g0, g1, x2d, rt)


def kernel(x, experts_inter, experts_out, router_w, router_b):
    x2d = x.reshape(T, HIDDEN)
    wr_t = router_w.T
    rb_bcast = jnp.broadcast_to(router_b[None, :], (8, NUM_EXPERTS))

    rt, cnt = _run_router(x2d, wr_t, rb_bcast)

    d0 = rt[:, 0].astype(jnp.int32)
    d1 = rt[:, 1].astype(jnp.int32)
    c0r = rt[:, 2].astype(jnp.int32).reshape(_NW, T // _NW // _CCH, _CCH)
    c1r = rt[:, 3].astype(jnp.int32).reshape(_NW, T // _NW // _CCH, _CCH)

    counts = cnt[0, :EC]
    slot_iota = lax.broadcasted_iota(jnp.float32, (EC, CAP), 1)
    vmask = (slot_iota < counts[:, None]).astype(jnp.float32).reshape(NROW, 1)

    tokp = _make_tokmap_sc()(d0, d1)
    buf = _make_dispatch_sc()(x2d, tokp)
    out_buf = _run_ffn(buf, vmask, experts_inter, experts_out)
    g0, g1 = _make_gather_sc()(out_buf, c0r, c1r)
    y = _run_combine(g0, g1, x2d, rt)
    return y.reshape(BATCH, SEQ, HIDDEN)
